# baseline (speedup 1.0000x reference)
import functools

import jax
import jax.numpy as jnp
from jax import lax
from jax.experimental import pallas as pl
from jax.experimental.pallas import tpu as pltpu

H = 16
DH = 64
D = 1024
BF = 16
SCALE = 1.0 / 8.0
NEG = -1e30


def _ln(x):
    m = jnp.mean(x, axis=-1, keepdims=True)
    v = jnp.mean((x - m) * (x - m), axis=-1, keepdims=True)
    return (x - m) / jnp.sqrt(v + 1e-5)


def _b16(x):
    return x.astype(jnp.bfloat16).astype(jnp.float32)


def _b16r(x):
    y = lax.bitcast_convert_type(x, jnp.int32)
    r = y + jnp.int32(0x7FFF) + ((y >> 16) & 1)
    return lax.bitcast_convert_type(r & jnp.int32(-65536), jnp.float32)


def _dot_hi(a, b):
    return jnp.dot(a, b, preferred_element_type=jnp.float32,
                   precision=lax.Precision.HIGHEST)


def _dot(a, b, hi=True):
    return jnp.dot(_b16r(a), _b16r(b), preferred_element_type=jnp.float32,
                   precision=lax.Precision.HIGHEST)


def _mm_body(x_ref, w_ref, o_ref, *, ln):
    x = x_ref[...]
    if ln:
        x = _ln(x)
    o_ref[...] = _dot(x, w_ref[...])


def _mm(x, w, ln=False, br=256):
    r, d_in = x.shape
    d_out = w.shape[1]
    return pl.pallas_call(
        functools.partial(_mm_body, ln=ln),
        grid=(r // br,),
        in_specs=[
            pl.BlockSpec((br, d_in), lambda i: (i, 0)),
            pl.BlockSpec((d_in, d_out), lambda i: (0, 0)),
        ],
        out_specs=pl.BlockSpec((br, d_out), lambda i: (i, 0)),
        out_shape=jax.ShapeDtypeStruct((r, d_out), jnp.float32),
        compiler_params=pltpu.CompilerParams(
            dimension_semantics=("parallel",)),
    )(x, w)


def _mm2_body(x_ref, w1_ref, w2_ref, o1_ref, o2_ref):
    x = x_ref[...]
    o1_ref[...] = _dot(x, w1_ref[...])
    o2_ref[...] = _dot(x, w2_ref[...])


def _mm2(x, w1, w2, br=256):
    r, d_in = x.shape
    d_out = w1.shape[1]
    return pl.pallas_call(
        _mm2_body,
        grid=(r // br,),
        in_specs=[
            pl.BlockSpec((br, d_in), lambda i: (i, 0)),
            pl.BlockSpec((d_in, d_out), lambda i: (0, 0)),
            pl.BlockSpec((d_in, d_out), lambda i: (0, 0)),
        ],
        out_specs=[
            pl.BlockSpec((br, d_out), lambda i: (i, 0)),
            pl.BlockSpec((br, d_out), lambda i: (i, 0)),
        ],
        out_shape=[
            jax.ShapeDtypeStruct((r, d_out), jnp.float32),
            jax.ShapeDtypeStruct((r, d_out), jnp.float32),
        ],
        compiler_params=pltpu.CompilerParams(
            dimension_semantics=("parallel",)),
    )(x, w1, w2)


def _ffn_body(x_ref, w1_ref, w2_ref, o_ref, *, hi):
    j = pl.program_id(1)
    x = x_ref[...]
    t = jax.nn.gelu(_dot(_ln(x), w1_ref[...], hi=hi))
    contrib = _dot(t, w2_ref[...], hi=hi)

    @pl.when(j == 0)
    def _():
        o_ref[...] = x + contrib

    @pl.when(j > 0)
    def _():
        o_ref[...] = o_ref[...] + contrib


def _ffn(x, w1, w2, br=256, fc=2048, hi=True):
    r, d = x.shape
    ff = w1.shape[1]
    return pl.pallas_call(
        functools.partial(_ffn_body, hi=hi),
        grid=(r // br, ff // fc),
        in_specs=[
            pl.BlockSpec((br, d), lambda i, j: (i, 0)),
            pl.BlockSpec((d, fc), lambda i, j: (0, j)),
            pl.BlockSpec((fc, d), lambda i, j: (j, 0)),
        ],
        out_specs=pl.BlockSpec((br, d), lambda i, j: (i, 0)),
        out_shape=jax.ShapeDtypeStruct((r, d), jnp.float32),
        compiler_params=pltpu.CompilerParams(
            dimension_semantics=("parallel", "arbitrary")),
    )(x, w1, w2)


def _agg_body(x_ref, cls_ref, wq_ref, wk_ref, wv_ref, wo_ref, o_ref, *, bg):
    rows = bg * BF
    x = x_ref[...].reshape(rows, D)
    xn = _ln(x)
    k = _dot(xn, wk_ref[...])
    v = _dot(xn, wv_ref[...])

    c = cls_ref[...]
    qa = _dot(_ln(c), wq_ref[...])

    k3 = k.reshape(rows, H, DH)
    s = jnp.sum(k3 * qa.reshape(1, H, DH), axis=2) * SCALE
    s = s.reshape(bg, BF, H)
    m = jnp.max(s, axis=1, keepdims=True)
    e = jnp.exp(s - m)
    a = e / jnp.sum(e, axis=1, keepdims=True)

    v4 = v.reshape(bg, BF, H, DH)
    out = jnp.sum(v4 * a[:, :, :, None], axis=1)
    out = out.reshape(bg, D)
    o_ref[...] = c + _dot(out, wo_ref[...])


def _agg(x_groups, cls2d, wq, wk, wv, wo, bg=8):
    g = x_groups.shape[0]
    return pl.pallas_call(
        functools.partial(_agg_body, bg=bg),
        grid=(g // bg,),
        in_specs=[
            pl.BlockSpec((bg, BF, D), lambda i: (i, 0, 0)),
            pl.BlockSpec((1, D), lambda i: (0, 0)),
            pl.BlockSpec((D, D), lambda i: (0, 0)),
            pl.BlockSpec((D, D), lambda i: (0, 0)),
            pl.BlockSpec((D, D), lambda i: (0, 0)),
            pl.BlockSpec((D, D), lambda i: (0, 0)),
        ],
        out_specs=pl.BlockSpec((bg, D), lambda i: (i, 0)),
        out_shape=jax.ShapeDtypeStruct((g, D), jnp.float32),
        compiler_params=pltpu.CompilerParams(
            dimension_semantics=("parallel",)),
    )(x_groups, cls2d, wq, wk, wv, wo)


def _select_body(qh_ref, k1t_ref, k2t_ref, sel_ref, *, bq):
    qh = qh_ref[0]

    msum1 = jnp.zeros((bq, BF), jnp.float32)
    for h in range(H):
        sl = slice(h * DH, (h + 1) * DH)
        s = _dot_hi(qh[:, sl], k1t_ref[0][sl, :]) * SCALE
        m = jnp.max(s, axis=-1, keepdims=True)
        e = jnp.exp(s - m)
        msum1 = msum1 + e / jnp.sum(e, axis=-1, keepdims=True)
    m1 = jnp.max(msum1, axis=-1, keepdims=True)
    colidx1 = lax.broadcasted_iota(jnp.int32, (bq, BF), 1)
    sel0 = jnp.min(jnp.where(msum1 == m1, colidx1, BF), axis=-1)

    n2 = BF * BF
    colg2 = lax.broadcasted_iota(jnp.int32, (bq, n2), 1) // BF
    mask2 = colg2 == sel0[:, None]
    msum2 = jnp.zeros((bq, n2), jnp.float32)
    for h in range(H):
        sl = slice(h * DH, (h + 1) * DH)
        s = _dot_hi(qh[:, sl], k2t_ref[0][sl, :]) * SCALE
        s = jnp.where(mask2, s, NEG)
        m = jnp.max(s, axis=-1, keepdims=True)
        e = jnp.exp(s - m) * mask2.astype(jnp.float32)
        msum2 = msum2 + e / jnp.maximum(
            jnp.sum(e, axis=-1, keepdims=True), 1e-30)
    m2 = jnp.max(msum2, axis=-1, keepdims=True)
    colidx2 = lax.broadcasted_iota(jnp.int32, (bq, n2), 1)
    selg = jnp.min(jnp.where(msum2 == m2, colidx2, n2), axis=-1)
    sel_ref[...] = selg.reshape(1, 1, bq)


def _select(qhb, k1t, k2t, bq):
    b, mm, d = qhb.shape
    ni = mm // bq
    return pl.pallas_call(
        functools.partial(_select_body, bq=bq),
        grid=(b, ni),
        in_specs=[
            pl.BlockSpec((1, bq, d), lambda b_, i: (b_, i, 0)),
            pl.BlockSpec((1, d, BF), lambda b_, i: (b_, 0, 0)),
            pl.BlockSpec((1, d, BF * BF), lambda b_, i: (b_, 0, 0)),
        ],
        out_specs=pl.BlockSpec((1, 1, bq), lambda b_, i: (b_ * ni + i, 0, 0)),
        out_shape=jax.ShapeDtypeStruct((b * ni, 1, bq), jnp.int32),
        compiler_params=pltpu.CompilerParams(
            dimension_semantics=("arbitrary", "arbitrary")),
    )(qhb, k1t, k2t)


def _readout_body(qh_ref, sel_ref, k3t_ref, v3_ref, o_ref, *, bq, ch):
    kc = pl.program_id(2)
    qh = qh_ref[0]
    selg = jnp.reshape(sel_ref[...], (bq,))

    colg = (kc * ch + lax.broadcasted_iota(jnp.int32, (bq, ch), 1)) // BF
    mask = colg == selg[:, None]
    maskf = mask.astype(jnp.float32)
    outs = []
    for h in range(H):
        sl = slice(h * DH, (h + 1) * DH)
        s = _dot(qh[:, sl], k3t_ref[0][sl, :]) * SCALE
        s = jnp.where(mask, s, NEG)
        m = jnp.max(s, axis=-1, keepdims=True)
        e = jnp.exp(s - m) * maskf
        p = e / jnp.maximum(jnp.sum(e, axis=-1, keepdims=True), 1e-30)
        outs.append(_dot(p, v3_ref[0][:, sl], hi=False))
    out = jnp.concatenate(outs, axis=1)

    @pl.when(kc == 0)
    def _():
        o_ref[0] = out

    @pl.when(kc > 0)
    def _():
        o_ref[0] = o_ref[0] + out


def _readout(qhb, sel, k3t, v3b, bq, ch=1024):
    b, mm, d = qhb.shape
    n_leaf = v3b.shape[1]
    ni = mm // bq
    kcn = n_leaf // ch
    return pl.pallas_call(
        functools.partial(_readout_body, bq=bq, ch=ch),
        grid=(b, ni, kcn),
        in_specs=[
            pl.BlockSpec((1, bq, d), lambda b_, i, kc: (b_, i, 0)),
            pl.BlockSpec((1, 1, bq), lambda b_, i, kc: (b_ * ni + i, 0, 0)),
            pl.BlockSpec((1, d, ch), lambda b_, i, kc: (b_, 0, kc)),
            pl.BlockSpec((1, ch, d), lambda b_, i, kc: (b_, kc, 0)),
        ],
        out_specs=pl.BlockSpec((1, bq, d), lambda b_, i, kc: (b_, i, 0)),
        out_shape=jax.ShapeDtypeStruct((b, mm, d), jnp.float32),
        compiler_params=pltpu.CompilerParams(
            dimension_semantics=("arbitrary", "arbitrary", "arbitrary")),
    )(qhb, sel, k3t, v3b)


def _mm_res_body(x_ref, w_ref, r_ref, o_ref):
    o_ref[...] = _dot(x_ref[...], w_ref[...], hi=False) + r_ref[...]


def _mm_res(x, w, res, br=256):
    r, d_in = x.shape
    d_out = w.shape[1]
    return pl.pallas_call(
        _mm_res_body,
        grid=(r // br,),
        in_specs=[
            pl.BlockSpec((br, d_in), lambda i: (i, 0)),
            pl.BlockSpec((d_in, d_out), lambda i: (0, 0)),
            pl.BlockSpec((br, d_out), lambda i: (i, 0)),
        ],
        out_specs=pl.BlockSpec((br, d_out), lambda i: (i, 0)),
        out_shape=jax.ShapeDtypeStruct((r, d_out), jnp.float32),
        compiler_params=pltpu.CompilerParams(
            dimension_semantics=("parallel",)),
    )(x, w, res)


def kernel(query_data, layer_data, cls, Wq_a, Wk_a, Wv_a, Wo_a, W1_a, W2_a,
           Wq, Wk, Wv, Wo, W1, W2):
    b, m, d = query_data.shape
    n = layer_data.shape[1]
    n_grp = n // BF
    cls2d = cls.reshape(1, d)

    g1 = _agg(layer_data.reshape(b * n_grp, BF, d),
              cls2d, Wq_a, Wk_a, Wv_a, Wo_a, bg=8)
    lvl2 = _ffn(g1, W1_a, W2_a, br=256)
    g2 = _agg(lvl2.reshape(b * n_grp // BF, BF, d),
              cls2d, Wq_a, Wk_a, Wv_a, Wo_a, bg=8)
    lvl1 = _ffn(g2, W1_a, W2_a, br=32)

    qh = _mm(query_data.reshape(b * m, d), Wq, ln=True, br=256)
    k3, v3 = _mm2(layer_data.reshape(b * n, d), Wk, Wv, br=256)
    k12 = _mm(jnp.concatenate([lvl1, lvl2], axis=0), Wk, ln=False, br=32)

    k1t = k12[:b * BF].reshape(b, BF, d).transpose(0, 2, 1)
    k2t = k12[b * BF:].reshape(b, n_grp, d).transpose(0, 2, 1)
    k3t = k3.reshape(b, n, d).transpose(0, 2, 1)
    v3b = v3.reshape(b, n, d)
    qhb = qh.reshape(b, m, d)

    bq = 512
    sel = _select(qhb, k1t, k2t, bq=bq)
    att_out = _readout(qhb, sel, k3t, v3b, bq=bq, ch=1024)
    ret_pre = _mm_res(att_out.reshape(b * m, d), Wo,
                      query_data.reshape(b * m, d), br=256)
    ret = _ffn(ret_pre, W1, W2, br=256, hi=False)
    return ret.reshape(b, m, d)


# speedup vs baseline: 3.5299x; 3.5299x over previous
import functools

import jax
import jax.numpy as jnp
from jax import lax
from jax.experimental import pallas as pl
from jax.experimental.pallas import tpu as pltpu

H = 16
DH = 64
D = 1024
BF = 16
SCALE = 1.0 / 8.0
NEG = -1e30


def _ln(x):
    m = jnp.mean(x, axis=-1, keepdims=True)
    v = jnp.mean((x - m) * (x - m), axis=-1, keepdims=True)
    return (x - m) / jnp.sqrt(v + 1e-5)


def _b16(x):
    return x.astype(jnp.bfloat16).astype(jnp.float32)


def _b16r(x):
    y = lax.bitcast_convert_type(x, jnp.int32)
    r = y + jnp.int32(0x7FFF) + ((y >> 16) & 1)
    return lax.bitcast_convert_type(r & jnp.int32(-65536), jnp.float32)


def _dot_hi(a, b):
    return jnp.dot(a, b, preferred_element_type=jnp.float32,
                   precision=lax.Precision.HIGHEST)


def _dot(a, b, hi=True):
    return jnp.dot(a.astype(jnp.bfloat16), b.astype(jnp.bfloat16),
                   preferred_element_type=jnp.float32)


def _mm_body(x_ref, w_ref, o_ref, *, ln):
    x = x_ref[...]
    if ln:
        x = _ln(x)
    o_ref[...] = _dot(x, w_ref[...])


def _mm(x, w, ln=False, br=256):
    r, d_in = x.shape
    d_out = w.shape[1]
    return pl.pallas_call(
        functools.partial(_mm_body, ln=ln),
        grid=(r // br,),
        in_specs=[
            pl.BlockSpec((br, d_in), lambda i: (i, 0)),
            pl.BlockSpec((d_in, d_out), lambda i: (0, 0)),
        ],
        out_specs=pl.BlockSpec((br, d_out), lambda i: (i, 0)),
        out_shape=jax.ShapeDtypeStruct((r, d_out), jnp.float32),
        compiler_params=pltpu.CompilerParams(
            dimension_semantics=("parallel",)),
    )(x, w)


def _mm2_body(x_ref, w1_ref, w2_ref, o1_ref, o2_ref):
    x = x_ref[...]
    o1_ref[...] = _dot(x, w1_ref[...])
    o2_ref[...] = _dot(x, w2_ref[...])


def _mm2(x, w1, w2, br=256):
    r, d_in = x.shape
    d_out = w1.shape[1]
    return pl.pallas_call(
        _mm2_body,
        grid=(r // br,),
        in_specs=[
            pl.BlockSpec((br, d_in), lambda i: (i, 0)),
            pl.BlockSpec((d_in, d_out), lambda i: (0, 0)),
            pl.BlockSpec((d_in, d_out), lambda i: (0, 0)),
        ],
        out_specs=[
            pl.BlockSpec((br, d_out), lambda i: (i, 0)),
            pl.BlockSpec((br, d_out), lambda i: (i, 0)),
        ],
        out_shape=[
            jax.ShapeDtypeStruct((r, d_out), jnp.float32),
            jax.ShapeDtypeStruct((r, d_out), jnp.float32),
        ],
        compiler_params=pltpu.CompilerParams(
            dimension_semantics=("parallel",)),
    )(x, w1, w2)


def _ffn_body(x_ref, w1_ref, w2_ref, o_ref, *, hi):
    j = pl.program_id(1)
    x = x_ref[...]
    t = jax.nn.gelu(_dot(_ln(x), w1_ref[...], hi=hi))
    contrib = _dot(t, w2_ref[...], hi=hi)

    @pl.when(j == 0)
    def _():
        o_ref[...] = x + contrib

    @pl.when(j > 0)
    def _():
        o_ref[...] = o_ref[...] + contrib


def _ffn(x, w1, w2, br=256, fc=2048, hi=True):
    r, d = x.shape
    ff = w1.shape[1]
    return pl.pallas_call(
        functools.partial(_ffn_body, hi=hi),
        grid=(r // br, ff // fc),
        in_specs=[
            pl.BlockSpec((br, d), lambda i, j: (i, 0)),
            pl.BlockSpec((d, fc), lambda i, j: (0, j)),
            pl.BlockSpec((fc, d), lambda i, j: (j, 0)),
        ],
        out_specs=pl.BlockSpec((br, d), lambda i, j: (i, 0)),
        out_shape=jax.ShapeDtypeStruct((r, d), jnp.float32),
        compiler_params=pltpu.CompilerParams(
            dimension_semantics=("parallel", "arbitrary")),
    )(x, w1, w2)


def _agg_body(x_ref, cls_ref, wq_ref, wk_ref, wv_ref, wo_ref, o_ref, *, bg):
    rows = bg * BF
    x = x_ref[...].reshape(rows, D)
    xn = _ln(x)
    k = _dot(xn, wk_ref[...])
    v = _dot(xn, wv_ref[...])

    c = cls_ref[...]
    qa = _dot(_ln(c), wq_ref[...])

    k3 = k.reshape(rows, H, DH)
    s = jnp.sum(k3 * qa.reshape(1, H, DH), axis=2) * SCALE
    s = s.reshape(bg, BF, H)
    m = jnp.max(s, axis=1, keepdims=True)
    e = jnp.exp(s - m)
    a = e / jnp.sum(e, axis=1, keepdims=True)

    v4 = v.reshape(bg, BF, H, DH)
    out = jnp.sum(v4 * a[:, :, :, None], axis=1)
    out = out.reshape(bg, D)
    o_ref[...] = c + _dot(out, wo_ref[...])


def _agg(x_groups, cls2d, wq, wk, wv, wo, bg=8):
    g = x_groups.shape[0]
    return pl.pallas_call(
        functools.partial(_agg_body, bg=bg),
        grid=(g // bg,),
        in_specs=[
            pl.BlockSpec((bg, BF, D), lambda i: (i, 0, 0)),
            pl.BlockSpec((1, D), lambda i: (0, 0)),
            pl.BlockSpec((D, D), lambda i: (0, 0)),
            pl.BlockSpec((D, D), lambda i: (0, 0)),
            pl.BlockSpec((D, D), lambda i: (0, 0)),
            pl.BlockSpec((D, D), lambda i: (0, 0)),
        ],
        out_specs=pl.BlockSpec((bg, D), lambda i: (i, 0)),
        out_shape=jax.ShapeDtypeStruct((g, D), jnp.float32),
        compiler_params=pltpu.CompilerParams(
            dimension_semantics=("parallel",)),
    )(x_groups, cls2d, wq, wk, wv, wo)


def _select_body(qh_ref, k1t_ref, k2t_ref, sel_ref, *, bq):
    qh = qh_ref[0]

    msum1 = jnp.zeros((bq, BF), jnp.float32)
    for h in range(H):
        sl = slice(h * DH, (h + 1) * DH)
        s = _dot_hi(qh[:, sl], k1t_ref[0][sl, :]) * SCALE
        m = jnp.max(s, axis=-1, keepdims=True)
        e = jnp.exp(s - m)
        msum1 = msum1 + e / jnp.sum(e, axis=-1, keepdims=True)
    m1 = jnp.max(msum1, axis=-1, keepdims=True)
    colidx1 = lax.broadcasted_iota(jnp.int32, (bq, BF), 1)
    sel0 = jnp.min(jnp.where(msum1 == m1, colidx1, BF), axis=-1)

    n2 = BF * BF
    colg2 = lax.broadcasted_iota(jnp.int32, (bq, n2), 1) // BF
    mask2 = colg2 == sel0[:, None]
    msum2 = jnp.zeros((bq, n2), jnp.float32)
    for h in range(H):
        sl = slice(h * DH, (h + 1) * DH)
        s = _dot_hi(qh[:, sl], k2t_ref[0][sl, :]) * SCALE
        s = jnp.where(mask2, s, NEG)
        m = jnp.max(s, axis=-1, keepdims=True)
        e = jnp.exp(s - m) * mask2.astype(jnp.float32)
        msum2 = msum2 + e / jnp.maximum(
            jnp.sum(e, axis=-1, keepdims=True), 1e-30)
    m2 = jnp.max(msum2, axis=-1, keepdims=True)
    colidx2 = lax.broadcasted_iota(jnp.int32, (bq, n2), 1)
    selg = jnp.min(jnp.where(msum2 == m2, colidx2, n2), axis=-1)
    sel_ref[...] = selg.reshape(1, 1, bq)


def _select(qhb, k1t, k2t, bq):
    b, mm, d = qhb.shape
    ni = mm // bq
    return pl.pallas_call(
        functools.partial(_select_body, bq=bq),
        grid=(b, ni),
        in_specs=[
            pl.BlockSpec((1, bq, d), lambda b_, i: (b_, i, 0)),
            pl.BlockSpec((1, d, BF), lambda b_, i: (b_, 0, 0)),
            pl.BlockSpec((1, d, BF * BF), lambda b_, i: (b_, 0, 0)),
        ],
        out_specs=pl.BlockSpec((1, 1, bq), lambda b_, i: (b_ * ni + i, 0, 0)),
        out_shape=jax.ShapeDtypeStruct((b * ni, 1, bq), jnp.int32),
        compiler_params=pltpu.CompilerParams(
            dimension_semantics=("arbitrary", "arbitrary")),
    )(qhb, k1t, k2t)


def _readout_body(qh_ref, sel_ref, k3t_ref, v3_ref, o_ref, *, bq, ch):
    kc = pl.program_id(2)
    qh = qh_ref[0]
    selg = jnp.reshape(sel_ref[...], (bq,))

    colg = (kc * ch + lax.broadcasted_iota(jnp.int32, (bq, ch), 1)) // BF
    mask = colg == selg[:, None]
    maskf = mask.astype(jnp.float32)
    outs = []
    for h in range(H):
        sl = slice(h * DH, (h + 1) * DH)
        s = _dot(qh[:, sl], k3t_ref[0][sl, :]) * SCALE
        s = jnp.where(mask, s, NEG)
        m = jnp.max(s, axis=-1, keepdims=True)
        e = jnp.exp(s - m) * maskf
        p = e / jnp.maximum(jnp.sum(e, axis=-1, keepdims=True), 1e-30)
        outs.append(_dot(p, v3_ref[0][:, sl], hi=False))
    out = jnp.concatenate(outs, axis=1)

    @pl.when(kc == 0)
    def _():
        o_ref[0] = out

    @pl.when(kc > 0)
    def _():
        o_ref[0] = o_ref[0] + out


def _readout(qhb, sel, k3t, v3b, bq, ch=1024):
    b, mm, d = qhb.shape
    n_leaf = v3b.shape[1]
    ni = mm // bq
    kcn = n_leaf // ch
    return pl.pallas_call(
        functools.partial(_readout_body, bq=bq, ch=ch),
        grid=(b, ni, kcn),
        in_specs=[
            pl.BlockSpec((1, bq, d), lambda b_, i, kc: (b_, i, 0)),
            pl.BlockSpec((1, 1, bq), lambda b_, i, kc: (b_ * ni + i, 0, 0)),
            pl.BlockSpec((1, d, ch), lambda b_, i, kc: (b_, 0, kc)),
            pl.BlockSpec((1, ch, d), lambda b_, i, kc: (b_, kc, 0)),
        ],
        out_specs=pl.BlockSpec((1, bq, d), lambda b_, i, kc: (b_, i, 0)),
        out_shape=jax.ShapeDtypeStruct((b, mm, d), jnp.float32),
        compiler_params=pltpu.CompilerParams(
            dimension_semantics=("arbitrary", "arbitrary", "arbitrary")),
    )(qhb, sel, k3t, v3b)


def _mm_res_body(x_ref, w_ref, r_ref, o_ref):
    o_ref[...] = _dot(x_ref[...], w_ref[...], hi=False) + r_ref[...]


def _mm_res(x, w, res, br=256):
    r, d_in = x.shape
    d_out = w.shape[1]
    return pl.pallas_call(
        _mm_res_body,
        grid=(r // br,),
        in_specs=[
            pl.BlockSpec((br, d_in), lambda i: (i, 0)),
            pl.BlockSpec((d_in, d_out), lambda i: (0, 0)),
            pl.BlockSpec((br, d_out), lambda i: (i, 0)),
        ],
        out_specs=pl.BlockSpec((br, d_out), lambda i: (i, 0)),
        out_shape=jax.ShapeDtypeStruct((r, d_out), jnp.float32),
        compiler_params=pltpu.CompilerParams(
            dimension_semantics=("parallel",)),
    )(x, w, res)


def kernel(query_data, layer_data, cls, Wq_a, Wk_a, Wv_a, Wo_a, W1_a, W2_a,
           Wq, Wk, Wv, Wo, W1, W2):
    b, m, d = query_data.shape
    n = layer_data.shape[1]
    n_grp = n // BF
    cls2d = cls.reshape(1, d)

    g1 = _agg(layer_data.reshape(b * n_grp, BF, d),
              cls2d, Wq_a, Wk_a, Wv_a, Wo_a, bg=8)
    lvl2 = _ffn(g1, W1_a, W2_a, br=256)
    g2 = _agg(lvl2.reshape(b * n_grp // BF, BF, d),
              cls2d, Wq_a, Wk_a, Wv_a, Wo_a, bg=8)
    lvl1 = _ffn(g2, W1_a, W2_a, br=32)

    qh = _mm(query_data.reshape(b * m, d), Wq, ln=True, br=256)
    k3, v3 = _mm2(layer_data.reshape(b * n, d), Wk, Wv, br=256)
    k12 = _mm(jnp.concatenate([lvl1, lvl2], axis=0), Wk, ln=False, br=32)

    k1t = k12[:b * BF].reshape(b, BF, d).transpose(0, 2, 1)
    k2t = k12[b * BF:].reshape(b, n_grp, d).transpose(0, 2, 1)
    k3t = k3.reshape(b, n, d).transpose(0, 2, 1)
    v3b = v3.reshape(b, n, d)
    qhb = qh.reshape(b, m, d)

    bq = 512
    sel = _select(qhb, k1t, k2t, bq=bq)
    att_out = _readout(qhb, sel, k3t, v3b, bq=bq, ch=1024)
    ret_pre = _mm_res(att_out.reshape(b * m, d), Wo,
                      query_data.reshape(b * m, d), br=256)
    ret = _ffn(ret_pre, W1, W2, br=256, hi=False)
    return ret.reshape(b, m, d)
